# Initial kernel scaffold; baseline (speedup 1.0000x reference)
import functools

import jax
import jax.numpy as jnp
from jax import lax
from jax.experimental import pallas as pl
from jax.experimental.pallas import tpu as pltpu
from jax.experimental.pallas import tpu_sc as plsc

N = 256
IN = 128
HC = 64
OUT = 10
E = 8192
D1 = N * HC
ETOT = E + N
NW = 32
EPW = 272
EPAD = NW * EPW
ACC_W = 80
CN = 256
NSTEP = D1 // CN

_HI = lax.Precision.HIGHEST


def _prep_body(x_ref, w0_ref, b0_ref, wl_ref, bl_ref, wr_ref, br_ref,
               xl_ref, xr_ref):
    y = jnp.dot(x_ref[...], w0_ref[...], preferred_element_type=jnp.float32,
                precision=_HI) + b0_ref[...]
    y = jnp.maximum(y, 0.0)
    xl_ref[...] = jnp.dot(y, wl_ref[...], preferred_element_type=jnp.float32,
                          precision=_HI) + bl_ref[...]
    xr_ref[...] = jnp.dot(y, wr_ref[...], preferred_element_type=jnp.float32,
                          precision=_HI) + br_ref[...]


def _prep(x, W0, b0, Wl, bl, Wr, br):
    return pl.pallas_call(
        _prep_body,
        out_shape=[jax.ShapeDtypeStruct((N, HC), jnp.float32),
                   jax.ShapeDtypeStruct((N, HC), jnp.float32)],
    )(x, W0, b0.reshape(1, HC), Wl, bl.reshape(1, HC), Wr, br.reshape(1, HC))


def _gat_body(src_hbm, dst_hbm, xl_hbm, xr_hbm, att_hbm, out_hbm,
              src_v, dst_v, xlg, xrg, att_vm, acc):
    cid = lax.axis_index("c")
    sid = lax.axis_index("s")
    wid = sid * 2 + cid
    base = wid * EPW

    pltpu.sync_copy(src_hbm.at[pl.ds(base, EPW)], src_v)
    pltpu.sync_copy(dst_hbm.at[pl.ds(base, EPW)], dst_v)
    pltpu.sync_copy(att_hbm, att_vm)

    for lo, sz in ((0, 128), (128, 128), (256, 16)):
        pltpu.sync_copy(xl_hbm.at[src_v.at[pl.ds(lo, sz)]],
                        xlg.at[pl.ds(lo, sz)])
        pltpu.sync_copy(xr_hbm.at[dst_v.at[pl.ds(lo, sz)]],
                        xrg.at[pl.ds(lo, sz)])

    zero16 = jnp.zeros((16,), jnp.float32)

    @pl.loop(0, N)
    def _(r):
        for c in range(ACC_W // 16):
            acc[r, pl.ds(c * 16, 16)] = zero16

    lane0 = (lax.iota(jnp.int32, (16,)) == 0).astype(jnp.float32)

    @pl.loop(0, EPW)
    def _(e):
        d = dst_v[e]
        logit_acc = zero16
        stash = []
        for fo in range(HC // 16):
            sl = pl.ds(fo * 16, 16)
            a = xlg[e, sl]
            b = xrg[e, sl]
            s_ = a + b
            s_ = jnp.where(s_ >= 0.0, s_, 0.2 * s_)
            logit_acc = logit_acc + att_vm[sl] * s_
            stash.append(a)
        logit = jnp.sum(logit_acc)
        valid = (base + e) < ETOT
        p = jnp.exp(jnp.full((16,), logit, jnp.float32))
        p = p * jnp.where(valid, 1.0, 0.0).astype(jnp.float32)
        for fo in range(HC // 16):
            sl = pl.ds(fo * 16, 16)
            acc[d, sl] = acc[d, sl] + p * stash[fo]
        sl = pl.ds(HC, 16)
        acc[d, sl] = acc[d, sl] + p * lane0

    pltpu.sync_copy(acc, out_hbm.at[wid])


def _gat_sc(src_pad, dst_pad, xl, xr, att_v):
    mesh = plsc.VectorSubcoreMesh(core_axis_name="c", subcore_axis_name="s")
    run = functools.partial(
        pl.kernel,
        out_type=jax.ShapeDtypeStruct((NW, N, ACC_W), jnp.float32),
        mesh=mesh,
        scratch_types=[
            pltpu.VMEM((EPW,), jnp.int32),
            pltpu.VMEM((EPW,), jnp.int32),
            pltpu.VMEM((EPW, HC), jnp.float32),
            pltpu.VMEM((EPW, HC), jnp.float32),
            pltpu.VMEM((HC,), jnp.float32),
            pltpu.VMEM((N, ACC_W), jnp.float32),
        ],
    )(_gat_body)
    return run(src_pad, dst_pad, xl, xr, att_v)


def _norm_body(parts_ref, gb_ref, v_ref):
    p = parts_ref[...]
    num = jnp.sum(p[:, :, :HC], axis=0)
    den = jnp.sum(p[:, :, HC:HC + 1], axis=0)
    v_ref[...] = num / den + gb_ref[...]


def _norm(parts, gb):
    return pl.pallas_call(
        _norm_body,
        out_shape=jax.ShapeDtypeStruct((N, HC), jnp.float32),
    )(parts, gb.reshape(1, HC))


def _mlp_body(v_ref, w1_ref, b1_ref, w2_ref, b2_ref, w3_ref, b3_ref,
              o_ref, h1_ref):
    i = pl.program_id(0)
    chunk = jnp.dot(v_ref[...], w1_ref[...], preferred_element_type=jnp.float32,
                    precision=_HI)
    h1_ref[:, pl.ds(i * CN, CN)] = jnp.maximum(chunk + b1_ref[...], 0.0)

    @pl.when(i == NSTEP - 1)
    def _():
        h2 = jnp.dot(h1_ref[...], w2_ref[...], preferred_element_type=jnp.float32,
                     precision=_HI) + b2_ref[...]
        h2 = jnp.maximum(h2, 0.0)
        o_ref[...] = jnp.dot(h2, w3_ref[...], preferred_element_type=jnp.float32,
                             precision=_HI) + b3_ref[...]


def _mlp(v1d, W1, b1, W2, b2, W3, b3):
    return pl.pallas_call(
        _mlp_body,
        grid=(NSTEP,),
        in_specs=[
            pl.BlockSpec((1, D1), lambda i: (0, 0)),
            pl.BlockSpec((D1, CN), lambda i: (0, i)),
            pl.BlockSpec((1, CN), lambda i: (0, i)),
            pl.BlockSpec((D1, HC), lambda i: (0, 0)),
            pl.BlockSpec((1, HC), lambda i: (0, 0)),
            pl.BlockSpec((HC, OUT), lambda i: (0, 0)),
            pl.BlockSpec((1, OUT), lambda i: (0, 0)),
        ],
        out_specs=pl.BlockSpec((1, OUT), lambda i: (0, 0)),
        out_shape=jax.ShapeDtypeStruct((1, OUT), jnp.float32),
        scratch_shapes=[pltpu.VMEM((1, D1), jnp.float32)],
        compiler_params=pltpu.CompilerParams(
            dimension_semantics=("arbitrary",)),
    )(v1d, W1, b1.reshape(1, D1), W2, b2.reshape(1, HC), W3,
      b3.reshape(1, OUT))


def kernel(x, edge_index, W0, b0, Wl, bl, Wr, br, att, gb, W1, b1, W2, b2,
           W3, b3):
    loops = jnp.arange(N, dtype=jnp.int32)
    pad = jnp.zeros((EPAD - ETOT,), jnp.int32)
    src_pad = jnp.concatenate([edge_index[:, 0], loops, pad])
    dst_pad = jnp.concatenate([edge_index[:, 1], loops, pad])

    xl, xr = _prep(x, W0, b0, Wl, bl, Wr, br)
    parts = _gat_sc(src_pad, dst_pad, xl, xr, att.reshape(HC))
    v2 = _norm(parts, gb)
    out = _mlp(v2.reshape(1, D1), W1, b1, W2, b2, W3, b3)
    return out.reshape(OUT)


# speedup vs baseline: 1.6103x; 1.6103x over previous
import functools

import jax
import jax.numpy as jnp
from jax import lax
from jax.experimental import pallas as pl
from jax.experimental.pallas import tpu as pltpu
from jax.experimental.pallas import tpu_sc as plsc

N = 256
IN = 128
HC = 64
OUT = 10
E = 8192
D1 = N * HC
ETOT = E + N
NW = 32
EPW = 272
EPAD = NW * EPW
ACC_W = 80
CN = 2048
KB = 2048
NN = D1 // CN
NK = D1 // KB
FW = 128

_HI = lax.Precision.HIGHEST


def _prep_body(x_ref, w0_ref, b0_ref, wl_ref, bl_ref, wr_ref, br_ref,
               xl_ref, xr_ref):
    xb = x_ref[...].astype(jnp.bfloat16)
    y = jnp.dot(xb, w0_ref[...].astype(jnp.bfloat16),
                preferred_element_type=jnp.float32) + b0_ref[...]
    y = jnp.maximum(y, 0.0).astype(jnp.bfloat16)
    xl_ref[...] = jnp.dot(y, wl_ref[...].astype(jnp.bfloat16),
                          preferred_element_type=jnp.float32) + bl_ref[...]
    xr_ref[...] = jnp.dot(y, wr_ref[...].astype(jnp.bfloat16),
                          preferred_element_type=jnp.float32) + br_ref[...]


def _prep(x, W0, b0, Wl, bl, Wr, br):
    return pl.pallas_call(
        _prep_body,
        out_shape=[jax.ShapeDtypeStruct((N, FW), jnp.float32),
                   jax.ShapeDtypeStruct((N, FW), jnp.float32)],
    )(x, W0, b0.reshape(1, HC), Wl, bl.reshape(1, FW), Wr, br.reshape(1, FW))


def _gat_body(src_hbm, dst_hbm, xl_hbm, xr_hbm, att_hbm, out_hbm,
              src_v, dst_v, xlt, xrt, att_vm, acc, sem):
    cid = lax.axis_index("c")
    sid = lax.axis_index("s")
    wid = sid * 2 + cid
    base = wid * EPW

    c1 = pltpu.async_copy(src_hbm.at[pl.ds(base, EPW)], src_v, sem)
    c2 = pltpu.async_copy(dst_hbm.at[pl.ds(base, EPW)], dst_v, sem)
    c3 = pltpu.async_copy(xl_hbm, xlt, sem)
    c4 = pltpu.async_copy(xr_hbm, xrt, sem)
    c5 = pltpu.async_copy(att_hbm, att_vm, sem)

    zero16 = jnp.zeros((16,), jnp.float32)

    @pl.loop(0, N)
    def _(r):
        for c in range(ACC_W // 16):
            acc[r, pl.ds(c * 16, 16)] = zero16

    for c in (c1, c2, c3, c4, c5):
        c.wait()

    lane0 = (lax.iota(jnp.int32, 16) == 0).astype(jnp.float32)

    @pl.loop(0, EPW // 16)
    def _(g):
        s16 = src_v[pl.ds(g * 16, 16)]
        d16 = dst_v[pl.ds(g * 16, 16)]
        las = []
        for j in range(16):
            s = s16[j]
            d = d16[j]
            la = zero16
            for fo in range(HC // 16):
                sl = pl.ds(fo * 16, 16)
                s_ = xlt[s, sl] + xrt[d, sl]
                s_ = jnp.maximum(s_, 0.2 * s_)
                la = la + att_vm[sl] * s_
            las.append(la)
        ps = []
        for j in range(16):
            logit = jnp.sum(las[j])
            p = jnp.exp(jnp.full((16,), logit, jnp.float32))
            valid = (base + g * 16 + j) < ETOT
            ps.append(p * jnp.where(valid, 1.0, 0.0).astype(jnp.float32))
        for j in range(16):
            s = s16[j]
            d = d16[j]
            for fo in range(HC // 16):
                sl = pl.ds(fo * 16, 16)
                acc[d, sl] = acc[d, sl] + ps[j] * xlt[s, sl]
            sl = pl.ds(HC, 16)
            acc[d, sl] = acc[d, sl] + ps[j] * lane0

    pltpu.sync_copy(acc, out_hbm.at[wid])


def _gat_sc(src_pad, dst_pad, xl, xr, att_v):
    mesh = plsc.VectorSubcoreMesh(core_axis_name="c", subcore_axis_name="s")
    run = functools.partial(
        pl.kernel,
        out_type=jax.ShapeDtypeStruct((NW, N, ACC_W), jnp.float32),
        mesh=mesh,
        scratch_types=[
            pltpu.VMEM((EPW,), jnp.int32),
            pltpu.VMEM((EPW,), jnp.int32),
            pltpu.VMEM((N, FW), jnp.float32),
            pltpu.VMEM((N, FW), jnp.float32),
            pltpu.VMEM((HC,), jnp.float32),
            pltpu.VMEM((N, ACC_W), jnp.float32),
            pltpu.SemaphoreType.DMA,
        ],
        compiler_params=pltpu.CompilerParams(needs_layout_passes=False),
    )(_gat_body)
    return run(src_pad, dst_pad, xl, xr, att_v)


def _norm_body(parts_ref, gb_ref, v_ref):
    p = parts_ref[...]
    num = jnp.sum(p[:, :, :HC], axis=0)
    den = jnp.sum(p[:, :, HC:HC + 1], axis=0)
    v_ref[...] = num / den + gb_ref[...]


def _norm(parts, gb):
    return pl.pallas_call(
        _norm_body,
        out_shape=jax.ShapeDtypeStruct((N, HC), jnp.float32),
    )(parts, gb.reshape(1, HC))


def _mlp_body(v_ref, w1_ref, b1_ref, w2_ref, b2_ref, w3_ref, b3_ref,
              o_ref, h2acc_ref, acc_ref):
    n = pl.program_id(0)
    k = pl.program_id(1)
    vb = v_ref[:, pl.ds(k * KB, KB)].astype(jnp.bfloat16)
    w1b = w1_ref[...].astype(jnp.bfloat16)
    part = jnp.dot(vb, w1b, preferred_element_type=jnp.float32)

    @pl.when(k == 0)
    def _():
        acc_ref[...] = part

    @pl.when(k > 0)
    def _():
        acc_ref[...] = acc_ref[...] + part

    @pl.when(k == NK - 1)
    def _():
        h1c = jnp.maximum(acc_ref[...] + b1_ref[...], 0.0)
        part2 = jnp.dot(h1c.astype(jnp.bfloat16),
                        w2_ref[...].astype(jnp.bfloat16),
                        preferred_element_type=jnp.float32)

        @pl.when(n == 0)
        def _():
            h2acc_ref[...] = part2

        @pl.when(n > 0)
        def _():
            h2acc_ref[...] = h2acc_ref[...] + part2

        @pl.when(n == NN - 1)
        def _():
            h2 = jnp.maximum(h2acc_ref[...] + b2_ref[...], 0.0)
            o_ref[...] = jnp.dot(h2.astype(jnp.bfloat16),
                                 w3_ref[...].astype(jnp.bfloat16),
                                 preferred_element_type=jnp.float32) + b3_ref[...]


def _mlp(v1d, W1, b1, W2, b2, W3, b3):
    return pl.pallas_call(
        _mlp_body,
        grid=(NN, NK),
        in_specs=[
            pl.BlockSpec((1, D1), lambda n, k: (0, 0)),
            pl.BlockSpec((KB, CN), lambda n, k: (k, n)),
            pl.BlockSpec((1, CN), lambda n, k: (0, n)),
            pl.BlockSpec((CN, HC), lambda n, k: (n, 0)),
            pl.BlockSpec((1, HC), lambda n, k: (0, 0)),
            pl.BlockSpec((HC, OUT), lambda n, k: (0, 0)),
            pl.BlockSpec((1, OUT), lambda n, k: (0, 0)),
        ],
        out_specs=pl.BlockSpec((1, OUT), lambda n, k: (0, 0)),
        out_shape=jax.ShapeDtypeStruct((1, OUT), jnp.float32),
        scratch_shapes=[pltpu.VMEM((1, HC), jnp.float32),
                        pltpu.VMEM((1, CN), jnp.float32)],
        compiler_params=pltpu.CompilerParams(
            dimension_semantics=("arbitrary", "arbitrary")),
    )(v1d, W1, b1.reshape(1, D1), W2, b2.reshape(1, HC), W3,
      b3.reshape(1, OUT))


def kernel(x, edge_index, W0, b0, Wl, bl, Wr, br, att, gb, W1, b1, W2, b2,
           W3, b3):
    loops = jnp.arange(N, dtype=jnp.int32)
    pad = jnp.zeros((EPAD - ETOT,), jnp.int32)
    src_pad = jnp.concatenate([edge_index[:, 0], loops, pad])
    dst_pad = jnp.concatenate([edge_index[:, 1], loops, pad])

    zpadw = jnp.zeros((HC, FW - HC), jnp.float32)
    zpadb = jnp.zeros((FW - HC,), jnp.float32)
    Wlp = jnp.concatenate([Wl, zpadw], axis=1)
    Wrp = jnp.concatenate([Wr, zpadw], axis=1)
    blp = jnp.concatenate([bl, zpadb])
    brp = jnp.concatenate([br, zpadb])
    xl, xr = _prep(x, W0, b0, Wlp, blp, Wrp, brp)
    parts = _gat_sc(src_pad, dst_pad, xl, xr, att.reshape(HC))
    v2 = _norm(parts, gb)
    out = _mlp(v2.reshape(1, D1), W1, b1, W2, b2, W3, b3)
    return out.reshape(OUT)


# speedup vs baseline: 1.6130x; 1.0016x over previous
import functools

import jax
import jax.numpy as jnp
from jax import lax
from jax.experimental import pallas as pl
from jax.experimental.pallas import tpu as pltpu
from jax.experimental.pallas import tpu_sc as plsc

N = 256
IN = 128
HC = 64
OUT = 10
E = 8192
D1 = N * HC
ETOT = E + N
NW = 32
EPW = 272
EPAD = NW * EPW
ACC_W = 80
CN = 2048
KB = 2048
NN = D1 // CN
NK = D1 // KB
FW = 128


def _prep_body(x_ref, w0_ref, b0_ref, wl_ref, bl_ref, wr_ref, br_ref,
               xl_ref, xr_ref):
    xb = x_ref[...].astype(jnp.bfloat16)
    y = jnp.dot(xb, w0_ref[...].astype(jnp.bfloat16),
                preferred_element_type=jnp.float32) + b0_ref[...]
    y = jnp.maximum(y, 0.0).astype(jnp.bfloat16)
    xl_ref[...] = jnp.dot(y, wl_ref[...].astype(jnp.bfloat16),
                          preferred_element_type=jnp.float32) + bl_ref[...]
    xr_ref[...] = jnp.dot(y, wr_ref[...].astype(jnp.bfloat16),
                          preferred_element_type=jnp.float32) + br_ref[...]


def _prep(x, W0, b0, Wl, bl, Wr, br):
    return pl.pallas_call(
        _prep_body,
        out_shape=[jax.ShapeDtypeStruct((N, FW), jnp.float32),
                   jax.ShapeDtypeStruct((N, FW), jnp.float32)],
    )(x, W0, b0.reshape(1, HC), Wl, bl.reshape(1, FW), Wr, br.reshape(1, FW))


def _gat_body(src_hbm, dst_hbm, xl_hbm, xr_hbm, att_hbm, out_hbm,
              src_v, dst_v, xlt, xrt, att_vm, acc, sem):
    cid = lax.axis_index("c")
    sid = lax.axis_index("s")
    wid = sid * 2 + cid
    base = wid * EPW

    c1 = pltpu.async_copy(src_hbm.at[pl.ds(base, EPW)], src_v, sem)
    c2 = pltpu.async_copy(dst_hbm.at[pl.ds(base, EPW)], dst_v, sem)
    c3 = pltpu.async_copy(xl_hbm, xlt, sem)
    c4 = pltpu.async_copy(xr_hbm, xrt, sem)
    c5 = pltpu.async_copy(att_hbm, att_vm, sem)

    zero16 = jnp.zeros((16,), jnp.float32)

    @pl.loop(0, N)
    def _(r):
        for c in range(ACC_W // 16):
            acc[r, pl.ds(c * 16, 16)] = zero16

    for c in (c1, c2, c3, c4, c5):
        c.wait()

    lane0 = (lax.iota(jnp.int32, 16) == 0).astype(jnp.float32)

    @pl.loop(0, EPW // 16)
    def _(g):
        s16 = src_v[pl.ds(g * 16, 16)]
        d16 = dst_v[pl.ds(g * 16, 16)]
        las = []
        for j in range(16):
            s = s16[j]
            d = d16[j]
            la = zero16
            for fo in range(HC // 16):
                sl = pl.ds(fo * 16, 16)
                s_ = xlt[s, sl] + xrt[d, sl]
                s_ = jnp.maximum(s_, 0.2 * s_)
                la = la + att_vm[sl] * s_
            las.append(la)
        ps = []
        for j in range(16):
            logit = jnp.sum(las[j])
            p = jnp.exp(jnp.full((16,), logit, jnp.float32))
            valid = (base + g * 16 + j) < ETOT
            ps.append(p * jnp.where(valid, 1.0, 0.0).astype(jnp.float32))
        for j in range(16):
            s = s16[j]
            d = d16[j]
            for fo in range(HC // 16):
                sl = pl.ds(fo * 16, 16)
                acc[d, sl] = acc[d, sl] + ps[j] * xlt[s, sl]
            sl = pl.ds(HC, 16)
            acc[d, sl] = acc[d, sl] + ps[j] * lane0

    pltpu.sync_copy(acc, out_hbm.at[wid])


def _gat_sc(src_pad, dst_pad, xl, xr, att_v):
    mesh = plsc.VectorSubcoreMesh(core_axis_name="c", subcore_axis_name="s")
    run = functools.partial(
        pl.kernel,
        out_type=jax.ShapeDtypeStruct((NW, N, ACC_W), jnp.float32),
        mesh=mesh,
        scratch_types=[
            pltpu.VMEM((EPW,), jnp.int32),
            pltpu.VMEM((EPW,), jnp.int32),
            pltpu.VMEM((N, FW), jnp.float32),
            pltpu.VMEM((N, FW), jnp.float32),
            pltpu.VMEM((HC,), jnp.float32),
            pltpu.VMEM((N, ACC_W), jnp.float32),
            pltpu.SemaphoreType.DMA,
        ],
        compiler_params=pltpu.CompilerParams(needs_layout_passes=False),
    )(_gat_body)
    return run(src_pad, dst_pad, xl, xr, att_v)


def _norm_body(parts_ref, gb_ref, v_ref):
    p = parts_ref[...]
    num = jnp.sum(p[:, :, :HC], axis=0)
    den = jnp.sum(p[:, :, HC:HC + 1], axis=0)
    v_ref[...] = num / den + gb_ref[...]


def _norm(parts, gb):
    return pl.pallas_call(
        _norm_body,
        out_shape=jax.ShapeDtypeStruct((N, HC), jnp.float32),
    )(parts, gb.reshape(1, HC))


def _mlp_body(v_ref, w1_ref, b1_ref, w2_ref, b2_ref, w3_ref, b3_ref,
              o_ref, h2acc_ref, acc_ref):
    n = pl.program_id(0)
    k = pl.program_id(1)
    vb = v_ref[:, pl.ds(k * KB, KB)].astype(jnp.bfloat16)
    w1b = w1_ref[...].astype(jnp.bfloat16)
    part = jnp.dot(vb, w1b, preferred_element_type=jnp.float32)

    @pl.when(k == 0)
    def _():
        acc_ref[...] = part

    @pl.when(k > 0)
    def _():
        acc_ref[...] = acc_ref[...] + part

    @pl.when(k == NK - 1)
    def _():
        h1c = jnp.maximum(acc_ref[...] + b1_ref[...], 0.0)
        part2 = jnp.dot(h1c.astype(jnp.bfloat16),
                        w2_ref[...].astype(jnp.bfloat16),
                        preferred_element_type=jnp.float32)

        @pl.when(n == 0)
        def _():
            h2acc_ref[...] = part2

        @pl.when(n > 0)
        def _():
            h2acc_ref[...] = h2acc_ref[...] + part2

        @pl.when(n == NN - 1)
        def _():
            h2 = jnp.maximum(h2acc_ref[...] + b2_ref[...], 0.0)
            o_ref[...] = jnp.dot(h2.astype(jnp.bfloat16),
                                 w3_ref[...].astype(jnp.bfloat16),
                                 preferred_element_type=jnp.float32) + b3_ref[...]


def _mlp(v1d, W1, b1, W2, b2, W3, b3):
    return pl.pallas_call(
        _mlp_body,
        grid=(NN, NK),
        in_specs=[
            pl.BlockSpec((1, D1), lambda n, k: (0, 0)),
            pl.BlockSpec((KB, CN), lambda n, k: (k, n)),
            pl.BlockSpec((1, CN), lambda n, k: (0, n)),
            pl.BlockSpec((CN, HC), lambda n, k: (n, 0)),
            pl.BlockSpec((1, HC), lambda n, k: (0, 0)),
            pl.BlockSpec((HC, OUT), lambda n, k: (0, 0)),
            pl.BlockSpec((1, OUT), lambda n, k: (0, 0)),
        ],
        out_specs=pl.BlockSpec((1, OUT), lambda n, k: (0, 0)),
        out_shape=jax.ShapeDtypeStruct((1, OUT), jnp.float32),
        scratch_shapes=[pltpu.VMEM((1, HC), jnp.float32),
                        pltpu.VMEM((1, CN), jnp.float32)],
        compiler_params=pltpu.CompilerParams(
            dimension_semantics=("arbitrary", "arbitrary")),
    )(v1d, W1, b1.reshape(1, D1), W2, b2.reshape(1, HC), W3,
      b3.reshape(1, OUT))


def kernel(x, edge_index, W0, b0, Wl, bl, Wr, br, att, gb, W1, b1, W2, b2,
           W3, b3):
    loops = jnp.arange(N, dtype=jnp.int32)
    pad = jnp.zeros((EPAD - ETOT,), jnp.int32)
    src_pad = jnp.concatenate([edge_index[:, 0], loops, pad])
    dst_pad = jnp.concatenate([edge_index[:, 1], loops, pad])

    zpadw = jnp.zeros((HC, FW - HC), jnp.float32)
    zpadb = jnp.zeros((FW - HC,), jnp.float32)
    Wlp = jnp.concatenate([Wl, zpadw], axis=1)
    Wrp = jnp.concatenate([Wr, zpadw], axis=1)
    blp = jnp.concatenate([bl, zpadb])
    brp = jnp.concatenate([br, zpadb])
    xl, xr = _prep(x, W0, b0, Wlp, blp, Wrp, brp)
    parts = _gat_sc(src_pad, dst_pad, xl, xr, att.reshape(HC))
    v2 = _norm(parts, gb)
    out = _mlp(v2.reshape(1, D1), W1, b1, W2, b2, W3, b3)
    return out.reshape(OUT)
